# baseline (speedup 1.0000x reference)
import functools

import jax
import jax.numpy as jnp
import numpy as np
from jax import lax
from jax.experimental import pallas as pl
from jax.experimental.pallas import tpu as pltpu
from jax.experimental.pallas import tpu_sc as plsc

D = 640
N = 100000
Q = 1024
KTOP = 3
NLAB = 12
NLAYER = 3
RATIO = 0.2

BN = 2048
NBLK = (N + BN - 1) // BN
RQ = 128
NEG = float("-inf")


def _bdot(a, b):
    return jnp.dot(a.astype(jnp.bfloat16), b.astype(jnp.bfloat16),
                   preferred_element_type=jnp.float32)


def _round_bf16(a):
    return a.astype(jnp.bfloat16).astype(jnp.float32)


def _search_body(q_ref, db_ref, dbt_ref, ts_ref, ti_ref, qn_ref, st_ref, it_ref):
    i = pl.program_id(0)

    @pl.when(i == 0)
    def _init():
        q = q_ref[...]
        nrm = jnp.sqrt(jnp.sum(q * q, axis=1, keepdims=True))
        qn_ref[...] = q / nrm
        st_ref[...] = jnp.full((Q, 4), NEG, jnp.float32)
        it_ref[...] = jnp.zeros((Q, 4), jnp.int32)

    db = db_ref[...]
    dbt_ref[...] = db.T
    s = jnp.dot(qn_ref[...].astype(jnp.bfloat16), db.astype(jnp.bfloat16),
                preferred_element_type=jnp.float32)
    col = lax.broadcasted_iota(jnp.int32, (Q, BN), 1)
    base = i * BN
    s = jnp.where(base + col < N, s, NEG)

    s1 = st_ref[:, 0:1]
    s2 = st_ref[:, 1:2]
    s3 = st_ref[:, 2:3]
    i1 = it_ref[:, 0:1]
    i2 = it_ref[:, 1:2]
    i3 = it_ref[:, 2:3]
    for t in range(KTOP):
        m = jnp.max(s, axis=1, keepdims=True)
        idxl = jnp.min(jnp.where(s == m, col, jnp.int32(2**30)),
                       axis=1, keepdims=True)
        if t < KTOP - 1:
            s = jnp.where(col == idxl, NEG, s)
        ci = idxl + base
        gt1 = m > s1
        gt2 = m > s2
        gt3 = m > s3
        ns1 = jnp.where(gt1, m, s1)
        ni1 = jnp.where(gt1, ci, i1)
        ns2 = jnp.where(gt1, s1, jnp.where(gt2, m, s2))
        ni2 = jnp.where(gt1, i1, jnp.where(gt2, ci, i2))
        ns3 = jnp.where(gt2, s2, jnp.where(gt3, m, s3))
        ni3 = jnp.where(gt2, i2, jnp.where(gt3, ci, i3))
        s1, s2, s3, i1, i2, i3 = ns1, ns2, ns3, ni1, ni2, ni3
    st_ref[:, 0:1] = s1
    st_ref[:, 1:2] = s2
    st_ref[:, 2:3] = s3
    it_ref[:, 0:1] = i1
    it_ref[:, 1:2] = i2
    it_ref[:, 2:3] = i3
    ts_ref[...] = jnp.concatenate([s1, s2, s3], axis=1)
    ti_ref[...] = jnp.concatenate([i1, i2, i3], axis=1)


def _search(queries, db_weight, interpret=False):
    return pl.pallas_call(
        _search_body,
        grid=(NBLK,),
        in_specs=[
            pl.BlockSpec((Q, D), lambda i: (0, 0)),
            pl.BlockSpec((D, BN), lambda i: (0, i)),
        ],
        out_specs=[
            pl.BlockSpec((BN, D), lambda i: (i, 0)),
            pl.BlockSpec((Q, KTOP), lambda i: (0, 0)),
            pl.BlockSpec((Q, KTOP), lambda i: (0, 0)),
        ],
        out_shape=[
            jax.ShapeDtypeStruct((NBLK * BN, D), jnp.float32),
            jax.ShapeDtypeStruct((Q, KTOP), jnp.float32),
            jax.ShapeDtypeStruct((Q, KTOP), jnp.int32),
        ],
        scratch_shapes=[
            pltpu.VMEM((Q, D), jnp.float32),
            pltpu.VMEM((Q, 4), jnp.float32),
            pltpu.VMEM((Q, 4), jnp.int32),
        ],
        compiler_params=pltpu.CompilerParams(
            dimension_semantics=("arbitrary",)),
        interpret=interpret,
    )(queries, db_weight)


def _gather(db_t, db_label, idx_flat):
    info = plsc.get_sparse_core_info()
    nw = info.num_cores * info.num_subcores
    b = Q * KTOP
    bpw = b // nw
    mesh = plsc.VectorSubcoreMesh(core_axis_name="c", subcore_axis_name="s")

    @functools.partial(
        pl.kernel, mesh=mesh,
        out_type=[jax.ShapeDtypeStruct((b, D), jnp.float32),
                  jax.ShapeDtypeStruct((b,), jnp.int32)],
        scratch_types=[pltpu.VMEM((bpw,), jnp.int32),
                       pltpu.VMEM((bpw, D), jnp.float32),
                       pltpu.VMEM((bpw,), jnp.int32),
                       pltpu.SemaphoreType.DMA],
    )
    def gk(table_hbm, lbl_hbm, idx_hbm, seq_out, lbl_out, idx_v, rows_v, lv, sem):
        wid = lax.axis_index("s") * info.num_cores + lax.axis_index("c")
        base = wid * bpw
        pltpu.sync_copy(idx_hbm.at[pl.ds(base, bpw)], idx_v)
        pltpu.async_copy(table_hbm.at[idx_v], rows_v, sem).wait()
        pltpu.sync_copy(rows_v, seq_out.at[pl.ds(base, bpw)])
        pltpu.async_copy(lbl_hbm.at[idx_v], lv, sem).wait()
        pltpu.sync_copy(lv, lbl_out.at[pl.ds(base, bpw)])

    return gk(db_t, db_label, idx_flat)


def _mea_body(seq_ref, q_ref, sc_ref, lb_ref, wq_ref, wk_ref, wv_ref, wo_ref,
              bq_ref, bk_ref, bv_ref, bo_ref, dw_ref, dbias_ref, ow_ref,
              ob_ref, o_ref):
    r = KTOP * RQ
    qb = q_ref[...]
    sc = sc_ref[...]
    lb = lb_ref[...]
    diota = lax.broadcasted_iota(jnp.int32, (KTOP, RQ, D), 2)
    cls3 = jnp.where(lb == diota, 1.0, 0.0) * sc
    h0 = cls3.reshape(r, D)
    h1 = jnp.concatenate([qb, qb, qb], axis=0)
    h2 = seq_ref[...].reshape(r, D)
    scale = 1.0 / np.sqrt(D // 8)
    h = [h0, h1, h2]
    for i in range(NLAYER):
        last = i == NLAYER - 1
        wq = wq_ref[i]
        wk = wk_ref[i]
        wv = wv_ref[i]
        wo = wo_ref[i]
        bq = bq_ref[i][None, :]
        bk = bk_ref[i][None, :]
        bv = bv_ref[i][None, :]
        bo = bo_ref[i][None, :]
        qs = [_bdot(h[t], wq) + bq for t in range(1 if last else 3)]
        ks = [_bdot(h[t], wk) + bk for t in range(3)]
        vs = [_bdot(h[t], wv) + bv for t in range(3)]
        hn = []
        for ti in range(1 if last else 3):
            qr = _round_bf16(qs[ti])
            kr = [_round_bf16(k) for k in ks]
            a = [jnp.sum(qr * kr[tj], axis=1, keepdims=True) * scale
                 for tj in range(3)]
            m = jnp.maximum(jnp.maximum(a[0], a[1]), a[2])
            e = [jnp.exp(x - m) for x in a]
            den = e[0] + e[1] + e[2]
            w = [_round_bf16(x / den) for x in e]
            vr = [_round_bf16(v) for v in vs]
            attn = w[0] * vr[0] + w[1] * vr[1] + w[2] * vr[2]
            hn.append(_bdot(attn, wo) + bo)
        h = hn
    x = jnp.tanh(_bdot(h[0], dw_ref[...]) + dbias_ref[...][None, :])
    lg = _bdot(x, ow_ref[...]) + ob_ref[...][None, :]
    mea = jnp.mean(lg.reshape(KTOP, RQ, NLAB), axis=0)
    liota = lax.broadcasted_iota(jnp.int32, (KTOP, RQ, NLAB), 2)
    agg = jnp.sum(jnp.where(lb == liota, 1.0, 0.0), axis=0)
    ret = agg / jnp.sum(agg, axis=1, keepdims=True)
    o_ref[...] = mea * (1.0 - RATIO) + ret * RATIO


def _mea(seqs3, queries, ts_j, lb_j, WQ, WK, WV, WO, bQ, bK, bV, bO,
         dense_w, dense_b, out_w, out_b, interpret=False):
    nblk = Q // RQ
    return pl.pallas_call(
        _mea_body,
        grid=(nblk,),
        in_specs=[
            pl.BlockSpec((KTOP, RQ, D), lambda s: (0, s, 0)),
            pl.BlockSpec((RQ, D), lambda s: (s, 0)),
            pl.BlockSpec((KTOP, RQ, 1), lambda s: (0, s, 0)),
            pl.BlockSpec((KTOP, RQ, 1), lambda s: (0, s, 0)),
            pl.BlockSpec((NLAYER, D, D), lambda s: (0, 0, 0)),
            pl.BlockSpec((NLAYER, D, D), lambda s: (0, 0, 0)),
            pl.BlockSpec((NLAYER, D, D), lambda s: (0, 0, 0)),
            pl.BlockSpec((NLAYER, D, D), lambda s: (0, 0, 0)),
            pl.BlockSpec((NLAYER, D), lambda s: (0, 0)),
            pl.BlockSpec((NLAYER, D), lambda s: (0, 0)),
            pl.BlockSpec((NLAYER, D), lambda s: (0, 0)),
            pl.BlockSpec((NLAYER, D), lambda s: (0, 0)),
            pl.BlockSpec((D, D), lambda s: (0, 0)),
            pl.BlockSpec((D,), lambda s: (0,)),
            pl.BlockSpec((D, NLAB), lambda s: (0, 0)),
            pl.BlockSpec((NLAB,), lambda s: (0,)),
        ],
        out_specs=pl.BlockSpec((RQ, NLAB), lambda s: (s, 0)),
        out_shape=jax.ShapeDtypeStruct((Q, NLAB), jnp.float32),
        compiler_params=pltpu.CompilerParams(
            dimension_semantics=("arbitrary",)),
        interpret=interpret,
    )(seqs3, queries, ts_j, lb_j, WQ, WK, WV, WO, bQ, bK, bV, bO,
      dense_w, dense_b, out_w, out_b)


def kernel(queries, db_weight, db_label, WQ, WK, WV, WO, bQ, bK, bV, bO,
           dense_w, dense_b, out_w, out_b):
    db_t, ts, ti = _search(queries, db_weight)
    idx_jm = ti.T.reshape(-1)
    seqs, lbls = _gather(db_t, db_label, idx_jm)
    seqs3 = seqs.reshape(KTOP, Q, D)
    ts_j = ts.T[:, :, None]
    lb_j = lbls.reshape(KTOP, Q)[:, :, None]
    return _mea(seqs3, queries, ts_j, lb_j, WQ, WK, WV, WO, bQ, bK, bV, bO,
                dense_w, dense_b, out_w, out_b)


# speedup vs baseline: 1.0241x; 1.0241x over previous
import functools

import jax
import jax.numpy as jnp
import numpy as np
from jax import lax
from jax.experimental import pallas as pl
from jax.experimental.pallas import tpu as pltpu
from jax.experimental.pallas import tpu_sc as plsc

D = 640
N = 100000
Q = 1024
KTOP = 3
NLAB = 12
NLAYER = 3
RATIO = 0.2

BN = 2048
NBLK = (N + BN - 1) // BN
RQ = 128
NEG = float("-inf")


def _bdot(a, b):
    return jnp.dot(a, b, preferred_element_type=jnp.float32)


def _round_bf16(a):
    return a.astype(jnp.bfloat16).astype(jnp.float32)


def _search_body(q_ref, db_ref, dbt_ref, ts_ref, ti_ref, qn_ref, st_ref, it_ref):
    i = pl.program_id(0)

    @pl.when(i == 0)
    def _init():
        q = q_ref[...]
        nrm = jnp.sqrt(jnp.sum(q * q, axis=1, keepdims=True))
        qn_ref[...] = q / nrm
        st_ref[...] = jnp.full((Q, 4), NEG, jnp.float32)
        it_ref[...] = jnp.zeros((Q, 4), jnp.int32)

    db = db_ref[...]
    dbt_ref[...] = db.T
    s = jnp.dot(qn_ref[...], db, preferred_element_type=jnp.float32)
    col = lax.broadcasted_iota(jnp.int32, (Q, BN), 1)
    base = i * BN
    s = jnp.where(base + col < N, s, NEG)

    s1 = st_ref[:, 0:1]
    s2 = st_ref[:, 1:2]
    s3 = st_ref[:, 2:3]
    i1 = it_ref[:, 0:1]
    i2 = it_ref[:, 1:2]
    i3 = it_ref[:, 2:3]
    for t in range(KTOP):
        m = jnp.max(s, axis=1, keepdims=True)
        idxl = jnp.min(jnp.where(s == m, col, jnp.int32(2**30)),
                       axis=1, keepdims=True)
        if t < KTOP - 1:
            s = jnp.where(col == idxl, NEG, s)
        ci = idxl + base
        gt1 = m > s1
        gt2 = m > s2
        gt3 = m > s3
        ns1 = jnp.where(gt1, m, s1)
        ni1 = jnp.where(gt1, ci, i1)
        ns2 = jnp.where(gt1, s1, jnp.where(gt2, m, s2))
        ni2 = jnp.where(gt1, i1, jnp.where(gt2, ci, i2))
        ns3 = jnp.where(gt2, s2, jnp.where(gt3, m, s3))
        ni3 = jnp.where(gt2, i2, jnp.where(gt3, ci, i3))
        s1, s2, s3, i1, i2, i3 = ns1, ns2, ns3, ni1, ni2, ni3
    st_ref[:, 0:1] = s1
    st_ref[:, 1:2] = s2
    st_ref[:, 2:3] = s3
    it_ref[:, 0:1] = i1
    it_ref[:, 1:2] = i2
    it_ref[:, 2:3] = i3
    ts_ref[...] = jnp.concatenate([s1, s2, s3], axis=1)
    ti_ref[...] = jnp.concatenate([i1, i2, i3], axis=1)


def _search(queries, db_weight, interpret=False):
    return pl.pallas_call(
        _search_body,
        grid=(NBLK,),
        in_specs=[
            pl.BlockSpec((Q, D), lambda i: (0, 0)),
            pl.BlockSpec((D, BN), lambda i: (0, i)),
        ],
        out_specs=[
            pl.BlockSpec((BN, D), lambda i: (i, 0)),
            pl.BlockSpec((Q, KTOP), lambda i: (0, 0)),
            pl.BlockSpec((Q, KTOP), lambda i: (0, 0)),
        ],
        out_shape=[
            jax.ShapeDtypeStruct((NBLK * BN, D), jnp.float32),
            jax.ShapeDtypeStruct((Q, KTOP), jnp.float32),
            jax.ShapeDtypeStruct((Q, KTOP), jnp.int32),
        ],
        scratch_shapes=[
            pltpu.VMEM((Q, D), jnp.float32),
            pltpu.VMEM((Q, 4), jnp.float32),
            pltpu.VMEM((Q, 4), jnp.int32),
        ],
        compiler_params=pltpu.CompilerParams(
            dimension_semantics=("arbitrary",)),
        interpret=interpret,
    )(queries, db_weight)


def _gather(db_t, db_label, idx_flat):
    info = plsc.get_sparse_core_info()
    nw = info.num_cores * info.num_subcores
    b = Q * KTOP
    bpw = b // nw
    mesh = plsc.VectorSubcoreMesh(core_axis_name="c", subcore_axis_name="s")

    @functools.partial(
        pl.kernel, mesh=mesh,
        out_type=[jax.ShapeDtypeStruct((b, D), jnp.float32),
                  jax.ShapeDtypeStruct((b,), jnp.int32)],
        scratch_types=[pltpu.VMEM((bpw,), jnp.int32),
                       pltpu.VMEM((bpw, D), jnp.float32),
                       pltpu.VMEM((bpw,), jnp.int32),
                       pltpu.SemaphoreType.DMA],
    )
    def gk(table_hbm, lbl_hbm, idx_hbm, seq_out, lbl_out, idx_v, rows_v, lv, sem):
        wid = lax.axis_index("s") * info.num_cores + lax.axis_index("c")
        base = wid * bpw
        pltpu.sync_copy(idx_hbm.at[pl.ds(base, bpw)], idx_v)
        pltpu.async_copy(table_hbm.at[idx_v], rows_v, sem).wait()
        pltpu.sync_copy(rows_v, seq_out.at[pl.ds(base, bpw)])
        pltpu.async_copy(lbl_hbm.at[idx_v], lv, sem).wait()
        pltpu.sync_copy(lv, lbl_out.at[pl.ds(base, bpw)])

    return gk(db_t, db_label, idx_flat)


def _mea_body(seq_ref, q_ref, sc_ref, lb_ref, wq_ref, wk_ref, wv_ref, wo_ref,
              bq_ref, bk_ref, bv_ref, bo_ref, dw_ref, dbias_ref, ow_ref,
              ob_ref, o_ref):
    r = KTOP * RQ
    qb = q_ref[...]
    sc = sc_ref[...]
    lb = lb_ref[...]
    diota = lax.broadcasted_iota(jnp.int32, (KTOP, RQ, D), 2)
    cls3 = jnp.where(lb == diota, 1.0, 0.0) * sc
    h0 = cls3.reshape(r, D)
    h1 = jnp.concatenate([qb, qb, qb], axis=0)
    h2 = seq_ref[...].reshape(r, D)
    scale = 1.0 / np.sqrt(D // 8)
    h = [h0, h1, h2]
    for i in range(NLAYER):
        last = i == NLAYER - 1
        wq = wq_ref[i]
        wk = wk_ref[i]
        wv = wv_ref[i]
        wo = wo_ref[i]
        bq = bq_ref[i][None, :]
        bk = bk_ref[i][None, :]
        bv = bv_ref[i][None, :]
        bo = bo_ref[i][None, :]
        qs = [_bdot(h[t], wq) + bq for t in range(1 if last else 3)]
        ks = [_bdot(h[t], wk) + bk for t in range(3)]
        vs = [_bdot(h[t], wv) + bv for t in range(3)]
        hn = []
        for ti in range(1 if last else 3):
            qr = _round_bf16(qs[ti])
            kr = [_round_bf16(k) for k in ks]
            a = [jnp.sum(qr * kr[tj], axis=1, keepdims=True) * scale
                 for tj in range(3)]
            m = jnp.maximum(jnp.maximum(a[0], a[1]), a[2])
            e = [jnp.exp(x - m) for x in a]
            den = e[0] + e[1] + e[2]
            w = [_round_bf16(x / den) for x in e]
            vr = [_round_bf16(v) for v in vs]
            attn = w[0] * vr[0] + w[1] * vr[1] + w[2] * vr[2]
            hn.append(_bdot(attn, wo) + bo)
        h = hn
    x = jnp.tanh(_bdot(h[0], dw_ref[...]) + dbias_ref[...][None, :])
    lg = _bdot(x, ow_ref[...]) + ob_ref[...][None, :]
    mea = jnp.mean(lg.reshape(KTOP, RQ, NLAB), axis=0)
    liota = lax.broadcasted_iota(jnp.int32, (KTOP, RQ, NLAB), 2)
    agg = jnp.sum(jnp.where(lb == liota, 1.0, 0.0), axis=0)
    ret = agg / jnp.sum(agg, axis=1, keepdims=True)
    o_ref[...] = mea * (1.0 - RATIO) + ret * RATIO


def _mea(seqs3, queries, ts_j, lb_j, WQ, WK, WV, WO, bQ, bK, bV, bO,
         dense_w, dense_b, out_w, out_b, interpret=False):
    nblk = Q // RQ
    return pl.pallas_call(
        _mea_body,
        grid=(nblk,),
        in_specs=[
            pl.BlockSpec((KTOP, RQ, D), lambda s: (0, s, 0)),
            pl.BlockSpec((RQ, D), lambda s: (s, 0)),
            pl.BlockSpec((KTOP, RQ, 1), lambda s: (0, s, 0)),
            pl.BlockSpec((KTOP, RQ, 1), lambda s: (0, s, 0)),
            pl.BlockSpec((NLAYER, D, D), lambda s: (0, 0, 0)),
            pl.BlockSpec((NLAYER, D, D), lambda s: (0, 0, 0)),
            pl.BlockSpec((NLAYER, D, D), lambda s: (0, 0, 0)),
            pl.BlockSpec((NLAYER, D, D), lambda s: (0, 0, 0)),
            pl.BlockSpec((NLAYER, D), lambda s: (0, 0)),
            pl.BlockSpec((NLAYER, D), lambda s: (0, 0)),
            pl.BlockSpec((NLAYER, D), lambda s: (0, 0)),
            pl.BlockSpec((NLAYER, D), lambda s: (0, 0)),
            pl.BlockSpec((D, D), lambda s: (0, 0)),
            pl.BlockSpec((D,), lambda s: (0,)),
            pl.BlockSpec((D, NLAB), lambda s: (0, 0)),
            pl.BlockSpec((NLAB,), lambda s: (0,)),
        ],
        out_specs=pl.BlockSpec((RQ, NLAB), lambda s: (s, 0)),
        out_shape=jax.ShapeDtypeStruct((Q, NLAB), jnp.float32),
        compiler_params=pltpu.CompilerParams(
            dimension_semantics=("arbitrary",)),
        interpret=interpret,
    )(seqs3, queries, ts_j, lb_j, WQ, WK, WV, WO, bQ, bK, bV, bO,
      dense_w, dense_b, out_w, out_b)


def kernel(queries, db_weight, db_label, WQ, WK, WV, WO, bQ, bK, bV, bO,
           dense_w, dense_b, out_w, out_b):
    db_t, ts, ti = _search(queries, db_weight)
    idx_jm = ti.T.reshape(-1)
    seqs, lbls = _gather(db_t, db_label, idx_jm)
    seqs3 = seqs.reshape(KTOP, Q, D)
    ts_j = ts.T[:, :, None]
    lb_j = lbls.reshape(KTOP, Q)[:, :, None]
    return _mea(seqs3, queries, ts_j, lb_j, WQ, WK, WV, WO, bQ, bK, bV, bO,
                dense_w, dense_b, out_w, out_b)


# speedup vs baseline: 1.2236x; 1.1947x over previous
import functools

import jax
import jax.numpy as jnp
import numpy as np
from jax import lax
from jax.experimental import pallas as pl
from jax.experimental.pallas import tpu as pltpu
from jax.experimental.pallas import tpu_sc as plsc

D = 640
N = 100000
Q = 1024
KTOP = 3
NLAB = 12
NLAYER = 3
RATIO = 0.2

BN = 2048
NBLK = (N + BN - 1) // BN
RQ = 128
NEG = float("-inf")


def _bdot(a, b):
    return jnp.dot(a, b, preferred_element_type=jnp.float32)


def _round_bf16(a):
    return a.astype(jnp.bfloat16).astype(jnp.float32)


def _search_body(q_ref, db_ref, dbt_ref, ts_ref, ti_ref, qn_ref, f_ref, c_ref):
    i = pl.program_id(0)

    @pl.when(i == 0)
    def _init():
        q = q_ref[...]
        nrm = jnp.sqrt(jnp.sum(q * q, axis=1, keepdims=True))
        qn_ref[...] = q / nrm
        f_ref[...] = jnp.full((KTOP, Q, 128), NEG, jnp.float32)
        c_ref[...] = jnp.zeros((KTOP, Q, 128), jnp.int32)

    db = db_ref[...]
    dbt_ref[...] = db.T
    qn = qn_ref[...]
    lane = lax.broadcasted_iota(jnp.int32, (Q, 128), 1)
    base = i * BN
    f1 = f_ref[0]
    f2 = f_ref[1]
    f3 = f_ref[2]
    c1 = c_ref[0]
    c2 = c_ref[1]
    c3 = c_ref[2]
    for g in range(BN // 128):
        v = jnp.dot(qn, db[:, g * 128:(g + 1) * 128],
                    preferred_element_type=jnp.float32)
        gb = base + g * 128
        v = jnp.where(lane < (N - gb), v, NEG)
        cc = lane + gb
        gt1 = v > f1
        gt2 = v > f2
        gt3 = v > f3
        nf3 = jnp.where(gt2, f2, jnp.where(gt3, v, f3))
        nc3 = jnp.where(gt2, c2, jnp.where(gt3, cc, c3))
        nf2 = jnp.where(gt1, f1, jnp.where(gt2, v, f2))
        nc2 = jnp.where(gt1, c1, jnp.where(gt2, cc, c2))
        nf1 = jnp.where(gt1, v, f1)
        nc1 = jnp.where(gt1, cc, c1)
        f1, f2, f3, c1, c2, c3 = nf1, nf2, nf3, nc1, nc2, nc3
    f_ref[0] = f1
    f_ref[1] = f2
    f_ref[2] = f3
    c_ref[0] = c1
    c_ref[1] = c2
    c_ref[2] = c3

    @pl.when(i == NBLK - 1)
    def _final():
        ff1, ff2, ff3 = f1, f2, f3
        cc1, cc2, cc3 = c1, c2, c3
        outs = []
        outi = []
        for t in range(KTOP):
            m = jnp.max(ff1, axis=1, keepdims=True)
            c = jnp.min(jnp.where(ff1 == m, cc1, jnp.int32(2**30)),
                        axis=1, keepdims=True)
            outs.append(m)
            outi.append(c)
            if t < KTOP - 1:
                hit = cc1 == c
                ff1 = jnp.where(hit, ff2, ff1)
                cc1 = jnp.where(hit, cc2, cc1)
                ff2 = jnp.where(hit, ff3, ff2)
                cc2 = jnp.where(hit, cc3, cc2)
                ff3 = jnp.where(hit, NEG, ff3)
        ts_ref[...] = jnp.concatenate(outs, axis=1)
        ti_ref[...] = jnp.concatenate(outi, axis=1)


def _search(queries, db_weight, interpret=False):
    return pl.pallas_call(
        _search_body,
        grid=(NBLK,),
        in_specs=[
            pl.BlockSpec((Q, D), lambda i: (0, 0)),
            pl.BlockSpec((D, BN), lambda i: (0, i)),
        ],
        out_specs=[
            pl.BlockSpec((BN, D), lambda i: (i, 0)),
            pl.BlockSpec((Q, KTOP), lambda i: (0, 0)),
            pl.BlockSpec((Q, KTOP), lambda i: (0, 0)),
        ],
        out_shape=[
            jax.ShapeDtypeStruct((NBLK * BN, D), jnp.float32),
            jax.ShapeDtypeStruct((Q, KTOP), jnp.float32),
            jax.ShapeDtypeStruct((Q, KTOP), jnp.int32),
        ],
        scratch_shapes=[
            pltpu.VMEM((Q, D), jnp.float32),
            pltpu.VMEM((KTOP, Q, 128), jnp.float32),
            pltpu.VMEM((KTOP, Q, 128), jnp.int32),
        ],
        compiler_params=pltpu.CompilerParams(
            dimension_semantics=("arbitrary",)),
        interpret=interpret,
    )(queries, db_weight)


def _gather(db_t, db_label, idx_flat):
    info = plsc.get_sparse_core_info()
    nw = info.num_cores * info.num_subcores
    b = Q * KTOP
    bpw = b // nw
    mesh = plsc.VectorSubcoreMesh(core_axis_name="c", subcore_axis_name="s")

    @functools.partial(
        pl.kernel, mesh=mesh,
        out_type=[jax.ShapeDtypeStruct((b, D), jnp.float32),
                  jax.ShapeDtypeStruct((b,), jnp.int32)],
        scratch_types=[pltpu.VMEM((bpw,), jnp.int32),
                       pltpu.VMEM((bpw, D), jnp.float32),
                       pltpu.VMEM((bpw,), jnp.int32),
                       pltpu.SemaphoreType.DMA],
    )
    def gk(table_hbm, lbl_hbm, idx_hbm, seq_out, lbl_out, idx_v, rows_v, lv, sem):
        wid = lax.axis_index("s") * info.num_cores + lax.axis_index("c")
        base = wid * bpw
        pltpu.sync_copy(idx_hbm.at[pl.ds(base, bpw)], idx_v)
        pltpu.async_copy(table_hbm.at[idx_v], rows_v, sem).wait()
        pltpu.sync_copy(rows_v, seq_out.at[pl.ds(base, bpw)])
        pltpu.async_copy(lbl_hbm.at[idx_v], lv, sem).wait()
        pltpu.sync_copy(lv, lbl_out.at[pl.ds(base, bpw)])

    return gk(db_t, db_label, idx_flat)


def _mea_body(seq_ref, q_ref, sc_ref, lb_ref, wq_ref, wk_ref, wv_ref, wo_ref,
              bq_ref, bk_ref, bv_ref, bo_ref, dw_ref, dbias_ref, ow_ref,
              ob_ref, o_ref):
    r = KTOP * RQ
    qb = q_ref[...]
    sc = sc_ref[...]
    lb = lb_ref[...]
    diota = lax.broadcasted_iota(jnp.int32, (KTOP, RQ, D), 2)
    cls3 = jnp.where(lb == diota, 1.0, 0.0) * sc
    h0 = cls3.reshape(r, D)
    h1 = jnp.concatenate([qb, qb, qb], axis=0)
    h2 = seq_ref[...].reshape(r, D)
    scale = 1.0 / np.sqrt(D // 8)
    h = [h0, h1, h2]
    for i in range(NLAYER):
        last = i == NLAYER - 1
        wq = wq_ref[i]
        wk = wk_ref[i]
        wv = wv_ref[i]
        wo = wo_ref[i]
        bq = bq_ref[i][None, :]
        bk = bk_ref[i][None, :]
        bv = bv_ref[i][None, :]
        bo = bo_ref[i][None, :]
        qs = [_bdot(h[t], wq) + bq for t in range(1 if last else 3)]
        ks = [_bdot(h[t], wk) + bk for t in range(3)]
        vs = [_bdot(h[t], wv) + bv for t in range(3)]
        hn = []
        for ti in range(1 if last else 3):
            qr = _round_bf16(qs[ti])
            kr = [_round_bf16(k) for k in ks]
            a = [jnp.sum(qr * kr[tj], axis=1, keepdims=True) * scale
                 for tj in range(3)]
            m = jnp.maximum(jnp.maximum(a[0], a[1]), a[2])
            e = [jnp.exp(x - m) for x in a]
            den = e[0] + e[1] + e[2]
            w = [_round_bf16(x / den) for x in e]
            vr = [_round_bf16(v) for v in vs]
            attn = w[0] * vr[0] + w[1] * vr[1] + w[2] * vr[2]
            hn.append(_bdot(attn, wo) + bo)
        h = hn
    x = jnp.tanh(_bdot(h[0], dw_ref[...]) + dbias_ref[...][None, :])
    lg = _bdot(x, ow_ref[...]) + ob_ref[...][None, :]
    mea = jnp.mean(lg.reshape(KTOP, RQ, NLAB), axis=0)
    liota = lax.broadcasted_iota(jnp.int32, (KTOP, RQ, NLAB), 2)
    agg = jnp.sum(jnp.where(lb == liota, 1.0, 0.0), axis=0)
    ret = agg / jnp.sum(agg, axis=1, keepdims=True)
    o_ref[...] = mea * (1.0 - RATIO) + ret * RATIO


def _mea(seqs3, queries, ts_j, lb_j, WQ, WK, WV, WO, bQ, bK, bV, bO,
         dense_w, dense_b, out_w, out_b, interpret=False):
    nblk = Q // RQ
    return pl.pallas_call(
        _mea_body,
        grid=(nblk,),
        in_specs=[
            pl.BlockSpec((KTOP, RQ, D), lambda s: (0, s, 0)),
            pl.BlockSpec((RQ, D), lambda s: (s, 0)),
            pl.BlockSpec((KTOP, RQ, 1), lambda s: (0, s, 0)),
            pl.BlockSpec((KTOP, RQ, 1), lambda s: (0, s, 0)),
            pl.BlockSpec((NLAYER, D, D), lambda s: (0, 0, 0)),
            pl.BlockSpec((NLAYER, D, D), lambda s: (0, 0, 0)),
            pl.BlockSpec((NLAYER, D, D), lambda s: (0, 0, 0)),
            pl.BlockSpec((NLAYER, D, D), lambda s: (0, 0, 0)),
            pl.BlockSpec((NLAYER, D), lambda s: (0, 0)),
            pl.BlockSpec((NLAYER, D), lambda s: (0, 0)),
            pl.BlockSpec((NLAYER, D), lambda s: (0, 0)),
            pl.BlockSpec((NLAYER, D), lambda s: (0, 0)),
            pl.BlockSpec((D, D), lambda s: (0, 0)),
            pl.BlockSpec((D,), lambda s: (0,)),
            pl.BlockSpec((D, NLAB), lambda s: (0, 0)),
            pl.BlockSpec((NLAB,), lambda s: (0,)),
        ],
        out_specs=pl.BlockSpec((RQ, NLAB), lambda s: (s, 0)),
        out_shape=jax.ShapeDtypeStruct((Q, NLAB), jnp.float32),
        compiler_params=pltpu.CompilerParams(
            dimension_semantics=("arbitrary",)),
        interpret=interpret,
    )(seqs3, queries, ts_j, lb_j, WQ, WK, WV, WO, bQ, bK, bV, bO,
      dense_w, dense_b, out_w, out_b)


def kernel(queries, db_weight, db_label, WQ, WK, WV, WO, bQ, bK, bV, bO,
           dense_w, dense_b, out_w, out_b):
    db_t, ts, ti = _search(queries, db_weight)
    idx_jm = ti.T.reshape(-1)
    seqs, lbls = _gather(db_t, db_label, idx_jm)
    seqs3 = seqs.reshape(KTOP, Q, D)
    ts_j = ts.T[:, :, None]
    lb_j = lbls.reshape(KTOP, Q)[:, :, None]
    return _mea(seqs3, queries, ts_j, lb_j, WQ, WK, WV, WO, bQ, bK, bV, bO,
                dense_w, dense_b, out_w, out_b)
